# baseline (speedup 1.0000x reference)
import functools

import jax
import jax.numpy as jnp
from jax import lax
from jax.experimental import pallas as pl
from jax.experimental.pallas import tpu as pltpu
from jax.experimental.pallas import tpu_sc as plsc

BATCH = 16384
EMBED = 64
NUM_CORES = 2
NUM_SUBCORES = 16
NUM_WORKERS = NUM_CORES * NUM_SUBCORES
B_PER_W = BATCH // NUM_WORKERS
CHUNK = 256
N_CHUNKS = B_PER_W // CHUNK

_mesh = plsc.VectorSubcoreMesh(core_axis_name="c", subcore_axis_name="s")


@functools.partial(
    pl.kernel,
    mesh=_mesh,
    out_type=jax.ShapeDtypeStruct((BATCH // 2, 2 * EMBED), jnp.float32),
    scratch_types=[
        pltpu.VMEM((B_PER_W,), jnp.int32),
        pltpu.VMEM((CHUNK,), jnp.int32),
        pltpu.VMEM((CHUNK, 2 * EMBED), jnp.float32),
        pltpu.VMEM((B_PER_W // 2, 2 * EMBED), jnp.float32),
        pltpu.SemaphoreType.DMA,
    ],
)
def _embed_lookup(idx_hbm, table2_hbm, out2_hbm, idx_v, idx2_v, rows_v, dst_v,
                  sem):
    wid = lax.axis_index("s") * NUM_CORES + lax.axis_index("c")
    base = pl.multiple_of(wid * B_PER_W, B_PER_W)
    pltpu.sync_copy(idx_hbm.at[pl.ds(base, B_PER_W)], idx_v)

    for chunk in range(N_CHUNKS):
        c_off = chunk * CHUNK
        for k in range(CHUNK // 16):
            idx2_v[pl.ds(k * 16, 16)] = idx_v[pl.ds(c_off + k * 16, 16)] >> 1
        pltpu.async_copy(table2_hbm.at[idx2_v], rows_v, sem).wait()

        def extract(n, carry):
            rvec = idx_v[pl.ds(c_off + ((n >> 4) << 4), 16)]
            par = jnp.take_along_axis(rvec & 1, jnp.broadcast_to(n & 15, (16,)),
                                      axis=0, mode="promise_in_bounds")
            parf = par.astype(jnp.float32)
            dst_row = (c_off + n) >> 1
            dst_col = ((c_off + n) & 1) * EMBED
            for c4 in range(EMBED // 16):
                v0 = rows_v[n, pl.ds(c4 * 16, 16)]
                v1 = rows_v[n, pl.ds(EMBED + c4 * 16, 16)]
                dst_v[dst_row, pl.ds(dst_col + c4 * 16, 16)] = (
                    v0 + parf * (v1 - v0))
            return carry

        lax.fori_loop(0, CHUNK, extract, 0)

    pltpu.sync_copy(
        dst_v,
        out2_hbm.at[pl.ds(pl.multiple_of(base // 2, B_PER_W // 2),
                          B_PER_W // 2)])


def kernel(node_ids, table):
    table2 = table.reshape(table.shape[0] // 2, 2 * EMBED)
    out2 = _embed_lookup(node_ids.astype(jnp.int32), table2)
    return out2.reshape(BATCH, EMBED)


# speedup vs baseline: 2.1440x; 2.1440x over previous
import functools

import jax
import jax.numpy as jnp
from jax import lax
from jax.experimental import pallas as pl
from jax.experimental.pallas import tpu as pltpu
from jax.experimental.pallas import tpu_sc as plsc

BATCH = 16384
EMBED = 64
TILE_ROWS = 8
NUM_CORES = 2
NUM_SUBCORES = 16
NUM_WORKERS = NUM_CORES * NUM_SUBCORES
B_PER_W = BATCH // NUM_WORKERS
CHUNK = 64
N_CHUNKS = B_PER_W // CHUNK

_mesh = plsc.VectorSubcoreMesh(core_axis_name="c", subcore_axis_name="s")


def _lane_scalar(vec, lane):
    splat = jnp.take_along_axis(
        vec, jnp.broadcast_to(lane, (16,)), axis=0, mode="promise_in_bounds")
    return jnp.max(splat)


@functools.partial(
    pl.kernel,
    mesh=_mesh,
    out_type=jax.ShapeDtypeStruct((BATCH // 2, 2 * EMBED), jnp.float32),
    scratch_types=[
        pltpu.VMEM((B_PER_W,), jnp.int32),
        pltpu.VMEM((CHUNK, TILE_ROWS, EMBED), jnp.float32),
        pltpu.VMEM((B_PER_W // 2, 2 * EMBED), jnp.float32),
        pltpu.SemaphoreType.DMA,
    ],
    compiler_params=pltpu.CompilerParams(needs_layout_passes=False),
)
def _embed_lookup(idx_hbm, table3_hbm, out2_hbm, idx_v, blocks_v, dst_v, sem):
    wid = lax.axis_index("s") * NUM_CORES + lax.axis_index("c")
    base = pl.multiple_of(wid * B_PER_W, B_PER_W)
    pltpu.sync_copy(idx_hbm.at[pl.ds(base, B_PER_W)], idx_v)

    for chunk in range(N_CHUNKS):
        c_off = chunk * CHUNK
        copies = []
        for n in range(CHUNK):
            if n % 16 == 0:
                rvec = idx_v[pl.ds(c_off + n, 16)]
            blk = _lane_scalar(rvec, n % 16) >> 3
            copies.append(pltpu.async_copy(
                table3_hbm.at[pl.ds(blk, 1)], blocks_v.at[pl.ds(n, 1)], sem))
        for c in copies:
            c.wait()

        def extract(n, carry):
            g = c_off + n
            rvec = idx_v[pl.ds((g >> 4) << 4, 16)]
            sidx = _lane_scalar(rvec, g & 15)
            sub = sidx & 7
            dst_row = g >> 1
            dst_col = (g & 1) * EMBED
            for c4 in range(EMBED // 16):
                dst_v[dst_row, pl.ds(dst_col + c4 * 16, 16)] = (
                    blocks_v[n, sub, pl.ds(c4 * 16, 16)])
            return carry

        lax.fori_loop(0, CHUNK, extract, 0)

    pltpu.sync_copy(
        dst_v,
        out2_hbm.at[pl.ds(pl.multiple_of(base // 2, B_PER_W // 2),
                          B_PER_W // 2)])


def kernel(node_ids, table):
    table3 = table.reshape(table.shape[0] // TILE_ROWS, TILE_ROWS, EMBED)
    out2 = _embed_lookup(node_ids.astype(jnp.int32), table3)
    return out2.reshape(BATCH, EMBED)
